# baseline (speedup 1.0000x reference)
import functools

import jax
import jax.numpy as jnp
from jax import lax
from jax.experimental import pallas as pl
from jax.experimental.pallas import tpu as pltpu
from jax.experimental.pallas import tpu_sc as plsc

NEG_INF = float("-inf")
BIG_I32 = 2**30

_SC_CORES = 2
_SC_SUBCORES = 16
_SC_WORKERS = _SC_CORES * _SC_SUBCORES


def _encode_body(x_ref, m_ref, w_ref, o_ref):
    feat = x_ref[...] * m_ref[...]
    q = lax.dot_general(
        feat, w_ref[...], (((1,), (0,)), ((), ())),
        preferred_element_type=jnp.float32,
        precision=lax.Precision.DEFAULT,
    )
    n = jnp.sqrt(jnp.sum(q * q, axis=1, keepdims=True))
    o_ref[...] = q / jnp.maximum(n, 1e-12)


def _extract_top(v, gidx, count):
    vals, idxs = [], []
    for _ in range(count):
        m = jnp.max(v, axis=1, keepdims=True)
        cidx = jnp.where(v == m, gidx, BIG_I32)
        s = jnp.min(cidx, axis=1, keepdims=True)
        v = jnp.where(cidx == s, NEG_INF, v)
        vals.append(m)
        idxs.append(s)
    return vals, idxs


def _topk_body(qn_ref, ts_ref, oi_ref, rv_ref, ri_ref, *, kt, nk, topk):
    k = pl.program_id(0)

    @pl.when(k == 0)
    def _():
        rv_ref[...] = jnp.full(rv_ref.shape, NEG_INF, jnp.float32)
        ri_ref[...] = jnp.full(ri_ref.shape, BIG_I32, jnp.int32)

    sim = lax.dot_general(
        qn_ref[...], ts_ref[...], (((1,), (0,)), ((), ())),
        preferred_element_type=jnp.float32,
        precision=lax.Precision.DEFAULT,
    )
    b = sim.shape[0]
    gidx = lax.broadcasted_iota(jnp.int32, sim.shape, 1) + k * kt
    vals, idxs = _extract_top(sim, gidx, topk)

    pad_v = jnp.full((b, 8 - topk), NEG_INF, jnp.float32)
    pad_i = jnp.full((b, 8 - topk), BIG_I32, jnp.int32)
    new_v = jnp.concatenate(vals + [pad_v], axis=1)
    new_i = jnp.concatenate(idxs + [pad_i], axis=1)

    cv = jnp.concatenate([rv_ref[...], new_v], axis=1)
    ci = jnp.concatenate([ri_ref[...], new_i], axis=1)
    mvals, midxs = _extract_top(cv, ci, topk)
    rv_ref[...] = jnp.concatenate(mvals + [pad_v], axis=1)
    ri_ref[...] = jnp.concatenate(midxs + [pad_i], axis=1)

    @pl.when(k == nk - 1)
    def _():
        oi_ref[...] = ri_ref[...]


def _encode(x2, m2, w, bb=512):
    b, ct = x2.shape
    d = w.shape[1]
    return pl.pallas_call(
        _encode_body,
        grid=(b // bb,),
        in_specs=[
            pl.BlockSpec((bb, ct), lambda i: (i, 0)),
            pl.BlockSpec((bb, ct), lambda i: (i, 0)),
            pl.BlockSpec((ct, d), lambda i: (0, 0)),
        ],
        out_specs=pl.BlockSpec((bb, d), lambda i: (i, 0)),
        out_shape=jax.ShapeDtypeStruct((b, d), jnp.float32),
    )(x2, m2, w)


def _topk_idx(qn, ts_t, kt=512, topk=5):
    b, d = qn.shape
    k = ts_t.shape[1]
    nk = k // kt
    idx8 = pl.pallas_call(
        functools.partial(_topk_body, kt=kt, nk=nk, topk=topk),
        grid=(nk,),
        in_specs=[
            pl.BlockSpec((b, d), lambda i: (0, 0)),
            pl.BlockSpec((d, kt), lambda i: (0, i)),
        ],
        out_specs=pl.BlockSpec((b, 8), lambda i: (0, 0)),
        out_shape=jax.ShapeDtypeStruct((b, 8), jnp.int32),
        scratch_shapes=[
            pltpu.VMEM((b, 8), jnp.float32),
            pltpu.VMEM((b, 8), jnp.int32),
        ],
    )(qn, ts_t)
    return idx8[:, :topk]


def _gather_sc(idx3, text_emb, ts_emb, tsr_flat, ch=32):
    nw, nch, _ = idx3.shape
    n = nw * nch * ch
    d1 = text_emb.shape[1]
    d2 = ts_emb.shape[1]
    d3 = tsr_flat.shape[1]
    per_w = nch * ch
    mesh = plsc.VectorSubcoreMesh(core_axis_name="c", subcore_axis_name="s")

    @functools.partial(
        pl.kernel,
        mesh=mesh,
        out_type=[
            jax.ShapeDtypeStruct((n, d1), jnp.float32),
            jax.ShapeDtypeStruct((n, d2), jnp.float32),
            jax.ShapeDtypeStruct((n, d3), jnp.float32),
        ],
        scratch_types=[
            pltpu.VMEM((nch, ch), jnp.int32),
            pltpu.VMEM((ch, d1), jnp.float32),
            pltpu.VMEM((ch, d2), jnp.float32),
            pltpu.VMEM((ch, d3), jnp.float32),
            pltpu.SemaphoreType.DMA,
        ],
    )
    def gather_kernel(idx_hbm, t1_hbm, t2_hbm, t3_hbm, o1_hbm, o2_hbm, o3_hbm,
                      idx_v, b1, b2, b3, sem):
        wid = lax.axis_index("s") * _SC_CORES + lax.axis_index("c")
        base = wid * per_w
        pltpu.sync_copy(idx_hbm.at[wid], idx_v)

        def body(c, carry):
            isl = idx_v.at[c]
            c1 = pltpu.async_copy(t1_hbm.at[isl], b1, sem)
            c2 = pltpu.async_copy(t2_hbm.at[isl], b2, sem)
            c3 = pltpu.async_copy(t3_hbm.at[isl], b3, sem)
            c1.wait()
            c2.wait()
            c3.wait()
            out_off = base + c * ch
            pltpu.sync_copy(b1, o1_hbm.at[pl.ds(out_off, ch)])
            pltpu.sync_copy(b2, o2_hbm.at[pl.ds(out_off, ch)])
            pltpu.sync_copy(b3, o3_hbm.at[pl.ds(out_off, ch)])
            return carry

        lax.fori_loop(0, nch, body, 0)

    return gather_kernel(idx3, text_emb, ts_emb, tsr_flat)


def kernel(x_enc, input_mask, W_enc, ts_emb, text_emb, timeseries, top_k):
    b, c, t = x_enc.shape
    k, d = ts_emb.shape
    tk = 5

    x2 = x_enc.reshape(b, c * t)
    m2 = jnp.tile(input_mask, (1, c)).astype(jnp.float32)
    qn = _encode(x2, m2, W_enc)

    topk_idx = _topk_idx(qn, ts_emb.T, kt=512, topk=tk)

    ch = 32
    flat_idx = topk_idx.reshape(-1)
    n = flat_idx.shape[0]
    per_w = n // _SC_WORKERS
    idx3 = flat_idx.reshape(_SC_WORKERS, per_w // ch, ch)
    tsr_flat = timeseries.reshape(k, c * t)
    g1, g2, g3 = _gather_sc(idx3, text_emb, ts_emb, tsr_flat, ch=ch)

    text_topk = g1.reshape(b, tk, d)
    ts_topk = g2.reshape(b, tk, d)
    timeseries_topk = g3.reshape(b, tk, c, t)
    return (text_topk, ts_topk, timeseries_topk)


# speedup vs baseline: 5.1014x; 5.1014x over previous
import functools

import jax
import jax.numpy as jnp
from jax import lax
from jax.experimental import pallas as pl
from jax.experimental.pallas import tpu as pltpu
from jax.experimental.pallas import tpu_sc as plsc

NEG_INF = float("-inf")
BIG_I32 = 2**30

_SC_CORES = 2
_SC_SUBCORES = 16
_SC_WORKERS = _SC_CORES * _SC_SUBCORES


def _encode_body(x_ref, m_ref, w_ref, o_ref, *, c):
    mf = m_ref[...].astype(jnp.float32)
    feat = x_ref[...] * jnp.concatenate([mf] * c, axis=1)
    q = lax.dot_general(
        feat, w_ref[...], (((1,), (0,)), ((), ())),
        preferred_element_type=jnp.float32,
        precision=lax.Precision.DEFAULT,
    )
    n = jnp.sqrt(jnp.sum(q * q, axis=1, keepdims=True))
    o_ref[...] = q / jnp.maximum(n, 1e-12)


def _extract_top(v, gidx, count):
    vals, idxs = [], []
    for _ in range(count):
        m = jnp.max(v, axis=1, keepdims=True)
        cidx = jnp.where(v == m, gidx, BIG_I32)
        s = jnp.min(cidx, axis=1, keepdims=True)
        v = jnp.where(cidx == s, NEG_INF, v)
        vals.append(m)
        idxs.append(s)
    return vals, idxs


def _topk_body(qn_ref, ts_ref, oi_ref, rv_ref, ri_ref, *, kt, nk, topk):
    k = pl.program_id(0)

    @pl.when(k == 0)
    def _():
        rv_ref[...] = jnp.full(rv_ref.shape, NEG_INF, jnp.float32)
        ri_ref[...] = jnp.full(ri_ref.shape, BIG_I32, jnp.int32)

    sim = lax.dot_general(
        qn_ref[...], ts_ref[...], (((1,), (1,)), ((), ())),
        preferred_element_type=jnp.float32,
        precision=lax.Precision.DEFAULT,
    )
    b = sim.shape[0]
    gidx = lax.broadcasted_iota(jnp.int32, sim.shape, 1) + k * kt
    vals, idxs = _extract_top(sim, gidx, topk)

    pad_v = jnp.full((b, 8 - topk), NEG_INF, jnp.float32)
    pad_i = jnp.full((b, 8 - topk), BIG_I32, jnp.int32)
    new_v = jnp.concatenate(vals + [pad_v], axis=1)
    new_i = jnp.concatenate(idxs + [pad_i], axis=1)

    cv = jnp.concatenate([rv_ref[...], new_v], axis=1)
    ci = jnp.concatenate([ri_ref[...], new_i], axis=1)
    mvals, midxs = _extract_top(cv, ci, topk)
    rv_ref[...] = jnp.concatenate(mvals + [pad_v], axis=1)
    ri_ref[...] = jnp.concatenate(midxs + [pad_i], axis=1)

    @pl.when(k == nk - 1)
    def _():
        oi_ref[...] = ri_ref[...]


def _encode(x2, mask, w, bb=512):
    b, ct = x2.shape
    t = mask.shape[1]
    c = ct // t
    d = w.shape[1]
    return pl.pallas_call(
        functools.partial(_encode_body, c=c),
        grid=(b // bb,),
        in_specs=[
            pl.BlockSpec((bb, ct), lambda i: (i, 0)),
            pl.BlockSpec((bb, t), lambda i: (i, 0)),
            pl.BlockSpec((ct, d), lambda i: (0, 0)),
        ],
        out_specs=pl.BlockSpec((bb, d), lambda i: (i, 0)),
        out_shape=jax.ShapeDtypeStruct((b, d), jnp.float32),
    )(x2, mask, w)


def _topk_idx(qn, ts_emb, kt=512, topk=5):
    b, d = qn.shape
    k = ts_emb.shape[0]
    nk = k // kt
    idx8 = pl.pallas_call(
        functools.partial(_topk_body, kt=kt, nk=nk, topk=topk),
        grid=(nk,),
        in_specs=[
            pl.BlockSpec((b, d), lambda i: (0, 0)),
            pl.BlockSpec((kt, d), lambda i: (i, 0)),
        ],
        out_specs=pl.BlockSpec((b, 8), lambda i: (0, 0)),
        out_shape=jax.ShapeDtypeStruct((b, 8), jnp.int32),
        scratch_shapes=[
            pltpu.VMEM((b, 8), jnp.float32),
            pltpu.VMEM((b, 8), jnp.int32),
        ],
    )(qn, ts_emb)
    return idx8[:, :topk]


def _gather_sc(idx3, text_emb, ts_emb, tsr_flat, ch=32):
    nw, nch, _ = idx3.shape
    n = nw * nch * ch
    d1 = text_emb.shape[1]
    d2 = ts_emb.shape[1]
    d3 = tsr_flat.shape[1]
    per_w = nch * ch
    mesh = plsc.VectorSubcoreMesh(core_axis_name="c", subcore_axis_name="s")

    @functools.partial(
        pl.kernel,
        mesh=mesh,
        out_type=[
            jax.ShapeDtypeStruct((n, d1), jnp.float32),
            jax.ShapeDtypeStruct((n, d2), jnp.float32),
            jax.ShapeDtypeStruct((n, d3), jnp.float32),
        ],
        scratch_types=[
            pltpu.VMEM((nch, ch), jnp.int32),
            pltpu.VMEM((ch, d1), jnp.float32),
            pltpu.VMEM((ch, d2), jnp.float32),
            pltpu.VMEM((ch, d3), jnp.float32),
            pltpu.SemaphoreType.DMA,
        ],
    )
    def gather_kernel(idx_hbm, t1_hbm, t2_hbm, t3_hbm, o1_hbm, o2_hbm, o3_hbm,
                      idx_v, b1, b2, b3, sem):
        wid = lax.axis_index("s") * _SC_CORES + lax.axis_index("c")
        base = wid * per_w
        pltpu.sync_copy(idx_hbm.at[wid], idx_v)

        def body(c, carry):
            isl = idx_v.at[c]
            c1 = pltpu.async_copy(t1_hbm.at[isl], b1, sem)
            c2 = pltpu.async_copy(t2_hbm.at[isl], b2, sem)
            c3 = pltpu.async_copy(t3_hbm.at[isl], b3, sem)
            c1.wait()
            c2.wait()
            c3.wait()
            out_off = base + c * ch
            pltpu.sync_copy(b1, o1_hbm.at[pl.ds(out_off, ch)])
            pltpu.sync_copy(b2, o2_hbm.at[pl.ds(out_off, ch)])
            pltpu.sync_copy(b3, o3_hbm.at[pl.ds(out_off, ch)])
            return carry

        lax.fori_loop(0, nch, body, 0)

    return gather_kernel(idx3, text_emb, ts_emb, tsr_flat)


def kernel(x_enc, input_mask, W_enc, ts_emb, text_emb, timeseries, top_k):
    b, c, t = x_enc.shape
    k, d = ts_emb.shape
    tk = 5

    x2 = x_enc.reshape(b, c * t)
    qn = _encode(x2, input_mask, W_enc)

    topk_idx = _topk_idx(qn, ts_emb.T, kt=512, topk=tk)

    ch = 32
    flat_idx = topk_idx.reshape(-1)
    n = flat_idx.shape[0]
    per_w = n // _SC_WORKERS
    idx3 = flat_idx.reshape(_SC_WORKERS, per_w // ch, ch)
    tsr_flat = timeseries.reshape(k, c * t)
    g1, g2, g3 = _gather_sc(idx3, text_emb, ts_emb, tsr_flat, ch=ch)

    text_topk = g1.reshape(b, tk, d)
    ts_topk = g2.reshape(b, tk, d)
    timeseries_topk = g3.reshape(b, tk, c, t)
    return (text_topk, ts_topk, timeseries_topk)
